# baseline (device time: 51905 ns/iter reference)
import jax
import jax.numpy as jnp
from jax import lax
from jax.experimental import pallas as pl
from jax.experimental.pallas import tpu as pltpu

N_DEV = 8
BLK = 64
F_HOPS = 4
B_HOPS = 3


def kernel(x, Wq, K_ext, V_ext, Wo):
    B, Sq, Dm = x.shape
    _, Skv, Hq, Dh = K_ext.shape
    HD = Hq * Dh
    nblk = Sq // BLK
    K2 = K_ext.reshape(B, Skv, HD)
    V2 = V_ext.reshape(B, Skv, HD)

    def body(x_ref, wq_ref, k_ref, v_ref, wo_ref, out_ref,
             fbuf, bbuf, fsend, frecv, bsend, brecv):
        my = lax.axis_index("i")
        left = lax.rem(my + N_DEV - 1, N_DEV)
        right = lax.rem(my + 1, N_DEV)

        barrier = pltpu.get_barrier_semaphore()
        for nbr in (left, right):
            pl.semaphore_signal(barrier, inc=1, device_id=(nbr,),
                                device_id_type=pl.DeviceIdType.MESH)
        pl.semaphore_wait(barrier, 2)

        for b in range(B):
            fbuf[0, 2 * b] = k_ref[b]
            fbuf[0, 2 * b + 1] = v_ref[b]

        NP = 2 * B

        def piece(buf, slot, p):
            return buf.at[slot, p]

        f_rdma = [
            [pltpu.make_async_remote_copy(
                src_ref=piece(fbuf, h, p), dst_ref=piece(fbuf, h + 1, p),
                send_sem=fsend.at[h, p], recv_sem=frecv.at[h, p],
                device_id=(right,), device_id_type=pl.DeviceIdType.MESH)
             for p in range(NP)]
            for h in range(F_HOPS)
        ]
        b_rdma = [
            [pltpu.make_async_remote_copy(
                src_ref=(piece(fbuf, 0, p) if h == 0
                         else piece(bbuf, h - 1, p)),
                dst_ref=piece(bbuf, h, p),
                send_sem=bsend.at[h, p], recv_sem=brecv.at[h, p],
                device_id=(left,), device_id_type=pl.DeviceIdType.MESH)
             for p in range(NP)]
            for h in range(B_HOPS)
        ]
        def f_send_p(h):
            return jnp.logical_and(my < N_DEV - 1, my >= h)

        def f_recv_p(h):
            return my > h

        def b_send_p(h):
            return lax.rem(my + h, N_DEV) <= B_HOPS - 1

        def b_recv_p(h):
            return lax.rem(my + h + 1, N_DEV) <= B_HOPS - 1

        for p in range(NP):
            pl.when(f_send_p(0))(lambda r=f_rdma[0][p]: r.start())
            pl.when(b_send_p(0))(lambda r=b_rdma[0][p]: r.start())

        q = [jnp.dot(x_ref[b], wq_ref[...],
                     preferred_element_type=jnp.float32) * 0.125
             for b in range(B)]

        qi_blk = lax.broadcasted_iota(jnp.int32, (Sq, Skv), 0) // BLK
        ki_blk = lax.broadcasted_iota(jnp.int32, (Sq, Skv), 1) // BLK

        m = [[jnp.full((Sq, 1), -1e30, jnp.float32)
              for _ in range(Hq)] for _ in range(B)]
        l = [[jnp.zeros((Sq, 1), jnp.float32)
              for _ in range(Hq)] for _ in range(B)]
        acc = [[jnp.zeros((Sq, Dh), jnp.float32)
                for _ in range(Hq)] for _ in range(B)]

        def process(buf, slot, origin):
            vis = (origin * nblk + ki_blk) <= (my * nblk + qi_blk)
            for b in range(B):
                kk = buf[slot, 2 * b]
                vv = buf[slot, 2 * b + 1]
                for h in range(Hq):
                    kbh = kk[:, h * Dh:(h + 1) * Dh]
                    vbh = vv[:, h * Dh:(h + 1) * Dh]
                    qbh = q[b][:, h * Dh:(h + 1) * Dh]
                    s = lax.dot_general(qbh, kbh, (((1,), (1,)), ((), ())),
                                        preferred_element_type=jnp.float32)
                    s = jnp.where(vis, s, -1e9)
                    m_new = jnp.maximum(m[b][h],
                                        jnp.max(s, axis=1, keepdims=True))
                    p = jnp.exp(s - m_new)
                    scale = jnp.exp(m[b][h] - m_new)
                    l[b][h] = l[b][h] * scale + jnp.sum(p, axis=1,
                                                        keepdims=True)
                    acc[b][h] = acc[b][h] * scale + jnp.dot(
                        p, vbh, preferred_element_type=jnp.float32)
                    m[b][h] = m_new

        process(fbuf, 0, my)

        for h in range(F_HOPS):
            for p in range(NP):
                pl.when(f_recv_p(h))(lambda r=f_rdma[h][p]: r.wait_recv())
                if h + 1 < F_HOPS:
                    pl.when(f_send_p(h + 1))(
                        lambda r=f_rdma[h + 1][p]: r.start())
            if h < B_HOPS:
                for p in range(NP):
                    pl.when(b_recv_p(h))(lambda r=b_rdma[h][p]: r.wait_recv())
                    if h + 1 < B_HOPS:
                        pl.when(b_send_p(h + 1))(
                            lambda r=b_rdma[h + 1][p]: r.start())
            process(fbuf, h + 1, lax.rem(my - h - 1 + N_DEV, N_DEV))
            if h < B_HOPS:
                process(bbuf, h, lax.rem(my + h + 1, N_DEV))

        for h in range(F_HOPS):
            for p in range(NP):
                pl.when(f_send_p(h))(lambda r=f_rdma[h][p]: r.wait_send())
        for h in range(B_HOPS):
            for p in range(NP):
                pl.when(b_send_p(h))(lambda r=b_rdma[h][p]: r.wait_send())

        for b in range(B):
            out_b = jnp.zeros((Sq, Dm), jnp.float32)
            for h in range(Hq):
                ctx = acc[b][h] / l[b][h]
                out_b = out_b + jnp.dot(ctx, wo_ref[h * Dh:(h + 1) * Dh, :],
                                        preferred_element_type=jnp.float32)
            out_ref[b] = out_b

    return pl.pallas_call(
        body,
        out_shape=jax.ShapeDtypeStruct((B, Sq, Dm), jnp.float32),
        in_specs=[pl.BlockSpec(memory_space=pltpu.VMEM)] * 5,
        out_specs=pl.BlockSpec(memory_space=pltpu.VMEM),
        scratch_shapes=[
            pltpu.VMEM((F_HOPS + 1, 2 * B, Skv, HD), jnp.float32),
            pltpu.VMEM((B_HOPS, 2 * B, Skv, HD), jnp.float32),
            pltpu.SemaphoreType.DMA((F_HOPS, 2 * B)),
            pltpu.SemaphoreType.DMA((F_HOPS, 2 * B)),
            pltpu.SemaphoreType.DMA((B_HOPS, 2 * B)),
            pltpu.SemaphoreType.DMA((B_HOPS, 2 * B)),
        ],
        compiler_params=pltpu.CompilerParams(collective_id=0),
    )(x, Wq, K2, V2, Wo)


# device time: 48996 ns/iter; 1.0594x vs baseline; 1.0594x over previous
import jax
import jax.numpy as jnp
from jax import lax
from jax.experimental import pallas as pl
from jax.experimental.pallas import tpu as pltpu

N_DEV = 8
BLK = 64
HOPS = 4
GA = (0, 1)
GB = (2, 3)


def kernel(x, Wq, K_ext, V_ext, Wo):
    B, Sq, Dm = x.shape
    _, Skv, Hq, Dh = K_ext.shape
    HD = Hq * Dh
    nblk = Sq // BLK
    K2 = K_ext.reshape(B, Skv, HD)
    V2 = V_ext.reshape(B, Skv, HD)

    def body(x_ref, wq_ref, k_ref, v_ref, wo_ref, out_ref,
             fbuf, bbuf, fsend, frecv, bsend, brecv):
        my = lax.axis_index("i")
        left = lax.rem(my + N_DEV - 1, N_DEV)
        right = lax.rem(my + 1, N_DEV)

        barrier = pltpu.get_barrier_semaphore()
        for nbr in (left, right):
            pl.semaphore_signal(barrier, inc=1, device_id=(nbr,),
                                device_id_type=pl.DeviceIdType.MESH)
        pl.semaphore_wait(barrier, 2)

        for b in range(B):
            fbuf[0, 2 * b] = k_ref[b]
            fbuf[0, 2 * b + 1] = v_ref[b]

        def piece(buf, slot, p):
            return buf.at[slot, p]

        f_rdma = [
            [pltpu.make_async_remote_copy(
                src_ref=piece(fbuf, h, p), dst_ref=piece(fbuf, h + 1, p),
                send_sem=fsend.at[h, p], recv_sem=frecv.at[h, p],
                device_id=(right,), device_id_type=pl.DeviceIdType.MESH)
             for p in range(4)]
            for h in range(HOPS)
        ]
        b_rdma = [
            [pltpu.make_async_remote_copy(
                src_ref=(piece(fbuf, 0, p) if h == 0
                         else piece(bbuf, h - 1, p)),
                dst_ref=piece(bbuf, h, p),
                send_sem=bsend.at[h, p], recv_sem=brecv.at[h, p],
                device_id=(left,), device_id_type=pl.DeviceIdType.MESH)
             for p in range(4)]
            for h in range(HOPS)
        ]

        def f_send_p(h):
            return jnp.logical_and(my < N_DEV - 1, my >= h)

        def f_recv_p(h):
            return my > h

        def b_send_p(h, grp):
            lim = 2 if grp is GA else 3
            return lax.rem(my + h, N_DEV) <= lim

        def b_recv_p(h, grp):
            lim = 2 if grp is GA else 3
            return lax.rem(my + h + 1, N_DEV) <= lim

        def f_hops(grp):
            return HOPS if grp is GA else HOPS - 1

        def b_hops(grp):
            return HOPS if grp is GB else HOPS - 1

        for grp in (GA, GB):
            for p in grp:
                pl.when(f_send_p(0))(lambda r=f_rdma[0][p]: r.start())
                pl.when(b_send_p(0, grp))(lambda r=b_rdma[0][p]: r.start())

        q = [jnp.dot(x_ref[b], wq_ref[...],
                     preferred_element_type=jnp.float32) * 0.125
             for b in range(B)]

        qi_blk = lax.broadcasted_iota(jnp.int32, (Sq, Skv), 0) // BLK
        ki_blk = lax.broadcasted_iota(jnp.int32, (Sq, Skv), 1) // BLK

        m = [[jnp.full((Sq, 1), -1e30, jnp.float32)
              for _ in range(Hq)] for _ in range(B)]
        l = [[jnp.zeros((Sq, 1), jnp.float32)
              for _ in range(Hq)] for _ in range(B)]
        acc = [[jnp.zeros((Sq, Dh), jnp.float32)
                for _ in range(Hq)] for _ in range(B)]

        def process(bufs, slots, origin):
            vis = (origin * nblk + ki_blk) <= (my * nblk + qi_blk)
            for b in range(B):
                kk = bufs[b][slots[b], 2 * b]
                vv = bufs[b][slots[b], 2 * b + 1]
                for h in range(Hq):
                    kbh = kk[:, h * Dh:(h + 1) * Dh]
                    vbh = vv[:, h * Dh:(h + 1) * Dh]
                    qbh = q[b][:, h * Dh:(h + 1) * Dh]
                    s = lax.dot_general(qbh, kbh, (((1,), (1,)), ((), ())),
                                        preferred_element_type=jnp.float32)
                    s = jnp.where(vis, s, -1e9)
                    m_new = jnp.maximum(m[b][h],
                                        jnp.max(s, axis=1, keepdims=True))
                    p = jnp.exp(s - m_new)
                    scale = jnp.exp(m[b][h] - m_new)
                    l[b][h] = l[b][h] * scale + jnp.sum(p, axis=1,
                                                        keepdims=True)
                    acc[b][h] = acc[b][h] * scale + jnp.dot(
                        p, vbh, preferred_element_type=jnp.float32)
                    m[b][h] = m_new

        process((fbuf, fbuf), (0, 0), my)

        for h in range(HOPS):
            for grp in (GA, GB):
                if h < f_hops(grp):
                    for p in grp:
                        pl.when(f_recv_p(h))(
                            lambda r=f_rdma[h][p]: r.wait_recv())
                        if h + 1 < f_hops(grp):
                            pl.when(f_send_p(h + 1))(
                                lambda r=f_rdma[h + 1][p]: r.start())
                if h < b_hops(grp):
                    for p in grp:
                        pl.when(b_recv_p(h, grp))(
                            lambda r=b_rdma[h][p]: r.wait_recv())
                        if h + 1 < b_hops(grp):
                            pl.when(b_send_p(h + 1, grp))(
                                lambda r=b_rdma[h + 1][p]: r.start())
            if h < HOPS - 1:
                process((fbuf, fbuf), (h + 1, h + 1),
                        lax.rem(my - h - 1 + N_DEV, N_DEV))
                process((bbuf, bbuf), (h, h), lax.rem(my + h + 1, N_DEV))

        process((fbuf, bbuf), (HOPS, HOPS - 1), lax.rem(my + 4, N_DEV))

        for grp in (GA, GB):
            for h in range(f_hops(grp)):
                for p in grp:
                    pl.when(f_send_p(h))(lambda r=f_rdma[h][p]: r.wait_send())
            for h in range(b_hops(grp)):
                for p in grp:
                    pl.when(b_send_p(h, grp))(
                        lambda r=b_rdma[h][p]: r.wait_send())

        for b in range(B):
            out_b = jnp.zeros((Sq, Dm), jnp.float32)
            for h in range(Hq):
                ctx = acc[b][h] / l[b][h]
                out_b = out_b + jnp.dot(ctx, wo_ref[h * Dh:(h + 1) * Dh, :],
                                        preferred_element_type=jnp.float32)
            out_ref[b] = out_b

    return pl.pallas_call(
        body,
        out_shape=jax.ShapeDtypeStruct((B, Sq, Dm), jnp.float32),
        in_specs=[pl.BlockSpec(memory_space=pltpu.VMEM)] * 5,
        out_specs=pl.BlockSpec(memory_space=pltpu.VMEM),
        scratch_shapes=[
            pltpu.VMEM((HOPS + 1, 2 * B, Skv, HD), jnp.float32),
            pltpu.VMEM((HOPS, 2 * B, Skv, HD), jnp.float32),
            pltpu.SemaphoreType.DMA((HOPS, 2 * B)),
            pltpu.SemaphoreType.DMA((HOPS, 2 * B)),
            pltpu.SemaphoreType.DMA((HOPS, 2 * B)),
            pltpu.SemaphoreType.DMA((HOPS, 2 * B)),
        ],
        compiler_params=pltpu.CompilerParams(collective_id=0),
    )(x, Wq, K2, V2, Wo)
